# baseline (speedup 1.0000x reference)
import functools

import jax
import jax.numpy as jnp
from jax import lax
from jax.experimental import pallas as pl
from jax.experimental.pallas import tpu as pltpu
from jax.experimental.pallas import tpu_sc as plsc

N = 10000
E = 320000
D_IN = 128
D1 = 16
DOUT = 40
DW = 16

NC, NS = 2, 16
NW = NC * NS
EPW = E // NW
B = 125
G = EPW // B
NP = 10240
RPT = NP // NS
ZB = 128
DEG_WIN = 4

_MESH = plsc.VectorSubcoreMesh(core_axis_name="c", subcore_axis_name="s")
_SC_PARAMS = pltpu.CompilerParams(use_tc_tiling_on_sc=False)


@functools.partial(
    pl.kernel,
    out_type=jax.ShapeDtypeStruct((NC * NP, DW), jnp.float32),
    mesh=_MESH,
    compiler_params=_SC_PARAMS,
    scratch_types=[
        pltpu.VMEM((G, B), jnp.int32),
        pltpu.VMEM((B, DW), jnp.float32),
        pltpu.VMEM((ZB, DW), jnp.float32),
        pltpu.VMEM_SHARED((NP, DW), jnp.float32),
        pltpu.SemaphoreType.DMA,
    ],
)
def _deg_kernel(dst_hbm, ones_hbm, zeros_hbm, deg_hbm,
                dst_v, ones_v, zeros_v, deg_sh, sem):
    c = lax.axis_index("c")
    s = lax.axis_index("s")
    wid = s * NC + c
    pltpu.sync_copy(dst_hbm.at[wid], dst_v)
    pltpu.sync_copy(ones_hbm, ones_v)
    pltpu.sync_copy(zeros_hbm, zeros_v)

    def zero_body(j, carry):
        pltpu.sync_copy(zeros_v, deg_sh.at[pl.ds(s * RPT + j * ZB, ZB)])
        return carry

    lax.fori_loop(0, RPT // ZB, zero_body, 0)
    plsc.subcore_barrier()

    def body(g, carry):
        pltpu.async_copy(ones_v, deg_sh.at[dst_v.at[g]], sem, add=True)

        @pl.when(g >= DEG_WIN)
        def _():
            pltpu.make_async_copy(ones_hbm, ones_v, sem).wait()

        return carry

    lax.fori_loop(0, G, body, 0)

    def drain_body(j, carry):
        pltpu.make_async_copy(ones_hbm, ones_v, sem).wait()
        return carry

    lax.fori_loop(0, DEG_WIN, drain_body, 0)
    plsc.subcore_barrier()
    pltpu.sync_copy(deg_sh.at[pl.ds(s * RPT, RPT)],
                    deg_hbm.at[pl.ds(c * NP + s * RPT, RPT)])


@functools.partial(
    pl.kernel,
    out_type=jax.ShapeDtypeStruct((NC * NP, D1), jnp.float32),
    mesh=_MESH,
    compiler_params=_SC_PARAMS,
    scratch_types=[
        pltpu.VMEM((G, B), jnp.int32),
        pltpu.VMEM((G, B), jnp.int32),
        pltpu.VMEM((B, D1), jnp.float32),
        pltpu.VMEM((B, D1), jnp.float32),
        pltpu.VMEM((ZB, D1), jnp.float32),
        pltpu.VMEM_SHARED((NP, D1), jnp.float32),
        pltpu.VMEM_SHARED((NP, D1), jnp.float32),
        pltpu.SemaphoreType.DMA,
        pltpu.SemaphoreType.DMA,
    ],
)
def _edge_kernel(src_hbm, dst_hbm, table_hbm, zeros_hbm, acc_hbm,
                 src_v, dst_v, rows0, rows1, zeros_v, acc_sh, table_sh,
                 sem0, sem1):
    c = lax.axis_index("c")
    s = lax.axis_index("s")
    wid = s * NC + c
    pltpu.sync_copy(src_hbm.at[wid], src_v)
    pltpu.sync_copy(dst_hbm.at[wid], dst_v)
    pltpu.sync_copy(zeros_hbm, zeros_v)
    pltpu.sync_copy(table_hbm.at[pl.ds(s * RPT, RPT)],
                    table_sh.at[pl.ds(s * RPT, RPT)])

    def zero_body(j, carry):
        pltpu.sync_copy(zeros_v, acc_sh.at[pl.ds(s * RPT + j * ZB, ZB)])
        return carry

    lax.fori_loop(0, RPT // ZB, zero_body, 0)
    plsc.subcore_barrier()

    def wait_gather(buf, sem):
        pltpu.make_async_copy(table_hbm.at[pl.ds(0, B)], buf, sem).wait()

    pltpu.async_copy(table_sh.at[src_v.at[0]], rows0, sem0)

    def body(p, carry):
        g0 = 2 * p
        pltpu.async_copy(table_sh.at[src_v.at[g0 + 1]], rows1, sem1)
        wait_gather(rows0, sem0)
        pltpu.sync_copy(rows0, acc_sh.at[dst_v.at[g0]], add=True)

        @pl.when(p < G // 2 - 1)
        def _():
            pltpu.async_copy(table_sh.at[src_v.at[g0 + 2]], rows0, sem0)

        wait_gather(rows1, sem1)
        pltpu.sync_copy(rows1, acc_sh.at[dst_v.at[g0 + 1]], add=True)
        return carry

    lax.fori_loop(0, G // 2, body, 0)
    plsc.subcore_barrier()
    pltpu.sync_copy(acc_sh.at[pl.ds(s * RPT, RPT)],
                    acc_hbm.at[pl.ds(c * NP + s * RPT, RPT)])


_BLK = 1024
_BLK8 = _BLK // 8
NPP = NP // 8


def _dis_packed(deg_ref):
    dg = deg_ref[...]
    return lax.rsqrt(dg[0] + dg[1] + 1.0)


def _mask_packed(i):
    prow = i * _BLK8 + lax.broadcasted_iota(jnp.int32, (_BLK8, 128), 0)
    sub = lax.broadcasted_iota(jnp.int32, (_BLK8, 128), 1) // D1
    return prow * 8 + sub < N


_deg_spec = pl.BlockSpec((NC, _BLK8, 128), lambda i: (0, i, 0))
_pk_spec = pl.BlockSpec((_BLK8, 128), lambda i: (i, 0))
_acc_spec = pl.BlockSpec((NC, _BLK8, 128), lambda i: (0, i, 0))


def _tc1_body(deg_ref, xpk_ref, wbig_ref, out_ref):
    h = jnp.dot(xpk_ref[...], wbig_ref[...], preferred_element_type=jnp.float32)
    out_ref[...] = jnp.where(_mask_packed(pl.program_id(0)),
                             h * _dis_packed(deg_ref), 0.0)


_tc1 = pl.pallas_call(
    _tc1_body,
    grid=(NPP // _BLK8,),
    in_specs=[
        _deg_spec,
        pl.BlockSpec((_BLK8, 8 * D_IN), lambda i: (i, 0)),
        pl.BlockSpec((8 * D_IN, 128), lambda i: (0, 0)),
    ],
    out_specs=_pk_spec,
    out_shape=jax.ShapeDtypeStruct((NPP, 128), jnp.float32),
)


def _tc2_body(deg_ref, acc_ref, h1s_ref, b1_ref, out_ref):
    dis = _dis_packed(deg_ref)
    acc = acc_ref[0] + acc_ref[1]
    x2 = jnp.maximum(dis * (acc + h1s_ref[...]) + b1_ref[...], 0.0)
    out_ref[...] = jnp.where(_mask_packed(pl.program_id(0)), x2 * dis, 0.0)


_tc2 = pl.pallas_call(
    _tc2_body,
    grid=(NPP // _BLK8,),
    in_specs=[
        _deg_spec,
        _acc_spec,
        _pk_spec,
        pl.BlockSpec((1, 128), lambda i: (0, 0)),
    ],
    out_specs=_pk_spec,
    out_shape=jax.ShapeDtypeStruct((NPP, 128), jnp.float32),
)


def _tc3_body(deg_ref, acc_ref, t2_ref, w2big_ref, b2_ref, sum_ref, rep_ref, out_ref):
    dis = _dis_packed(deg_ref)
    aggp = dis * (acc_ref[0] + acc_ref[1] + t2_ref[...])
    z = jnp.dot(aggp, w2big_ref[...], preferred_element_type=jnp.float32) + b2_ref[...]
    m = jnp.max(z, axis=1, keepdims=True)
    e = jnp.exp(z - m)
    sums = jnp.dot(e, sum_ref[...], preferred_element_type=jnp.float32)
    lse = jnp.log(sums) + m
    out_ref[...] = z - jnp.dot(lse, rep_ref[...], preferred_element_type=jnp.float32)


_tc3 = pl.pallas_call(
    _tc3_body,
    grid=(NPP // _BLK8,),
    in_specs=[
        _deg_spec,
        _acc_spec,
        _pk_spec,
        pl.BlockSpec((128, 8 * DOUT), lambda i: (0, 0)),
        pl.BlockSpec((1, 8 * DOUT), lambda i: (0, 0)),
        pl.BlockSpec((8 * DOUT, 8), lambda i: (0, 0)),
        pl.BlockSpec((8, 8 * DOUT), lambda i: (0, 0)),
    ],
    out_specs=pl.BlockSpec((_BLK8, 8 * DOUT), lambda i: (i, 0)),
    out_shape=jax.ShapeDtypeStruct((N // 8, 8 * DOUT), jnp.float32),
)


def kernel(x, edge_index, W1, b1, W2, b2):
    srcr = edge_index[0].reshape(NW, G, B)
    dstr = edge_index[1].reshape(NW, G, B)

    ones_b = jnp.ones((B, DW), jnp.float32)
    zeros_dw = jnp.zeros((ZB, DW), jnp.float32)
    zeros16 = jnp.zeros((ZB, D1), jnp.float32)

    xpk = x.reshape(N // 8, 8 * D_IN)
    w1big = jax.scipy.linalg.block_diag(*([W1] * 8))
    w2big = jax.scipy.linalg.block_diag(*([W2] * 8))
    b1p = jnp.tile(b1, 8).reshape(1, 128)
    b2p = jnp.tile(b2, 8).reshape(1, 8 * DOUT)
    sum_m = jnp.repeat(jnp.eye(8, dtype=jnp.float32), DOUT, axis=0)
    rep_m = sum_m.T

    degp = _deg_kernel(dstr, ones_b, zeros_dw).reshape(NC, NPP, 128)
    h1s = _tc1(degp, xpk, w1big)
    acc1 = _edge_kernel(srcr, dstr, h1s.reshape(NP, D1), zeros16).reshape(NC, NPP, 128)
    t2 = _tc2(degp, acc1, h1s, b1p)
    acc2 = _edge_kernel(srcr, dstr, t2.reshape(NP, D1), zeros16).reshape(NC, NPP, 128)
    return _tc3(degp, acc2, t2, w2big, b2p, sum_m, rep_m).reshape(N, DOUT)


# speedup vs baseline: 1.0694x; 1.0694x over previous
import functools

import jax
import jax.numpy as jnp
from jax import lax
from jax.experimental import pallas as pl
from jax.experimental.pallas import tpu as pltpu
from jax.experimental.pallas import tpu_sc as plsc

N = 10000
E = 320000
D_IN = 128
D1 = 16
DOUT = 40
DW = 16

NC, NS = 2, 16
NW = NC * NS
EPW = E // NW
B = 125
G = EPW // B
NP = 10240
RPT = NP // NS
ZB = 128
DEG_WIN = 4

_MESH = plsc.VectorSubcoreMesh(core_axis_name="c", subcore_axis_name="s")
_SC_PARAMS = pltpu.CompilerParams(use_tc_tiling_on_sc=False)


@functools.partial(
    pl.kernel,
    out_type=jax.ShapeDtypeStruct((NC * NP, DW), jnp.float32),
    mesh=_MESH,
    compiler_params=_SC_PARAMS,
    scratch_types=[
        pltpu.VMEM((G, B), jnp.int32),
        pltpu.VMEM((B, DW), jnp.float32),
        pltpu.VMEM((ZB, DW), jnp.float32),
        pltpu.VMEM_SHARED((NP, DW), jnp.float32),
        pltpu.SemaphoreType.DMA,
    ],
)
def _deg_kernel(edges_hbm, ones_hbm, zeros_hbm, deg_hbm,
                dst_v, ones_v, zeros_v, deg_sh, sem):
    c = lax.axis_index("c")
    s = lax.axis_index("s")
    wid = s * NC + c
    pltpu.sync_copy(edges_hbm.at[1, wid], dst_v)
    pltpu.sync_copy(ones_hbm, ones_v)
    pltpu.sync_copy(zeros_hbm, zeros_v)

    def zero_body(j, carry):
        pltpu.sync_copy(zeros_v, deg_sh.at[pl.ds(s * RPT + j * ZB, ZB)])
        return carry

    lax.fori_loop(0, RPT // ZB, zero_body, 0)
    plsc.subcore_barrier()

    def body(g, carry):
        pltpu.async_copy(ones_v, deg_sh.at[dst_v.at[g]], sem, add=True)

        @pl.when(g >= DEG_WIN)
        def _():
            pltpu.make_async_copy(ones_hbm, ones_v, sem).wait()

        return carry

    lax.fori_loop(0, G, body, 0)

    def drain_body(j, carry):
        pltpu.make_async_copy(ones_hbm, ones_v, sem).wait()
        return carry

    lax.fori_loop(0, DEG_WIN, drain_body, 0)
    plsc.subcore_barrier()
    pltpu.sync_copy(deg_sh.at[pl.ds(s * RPT, RPT)],
                    deg_hbm.at[pl.ds(c * NP + s * RPT, RPT)])


@functools.partial(
    pl.kernel,
    out_type=jax.ShapeDtypeStruct((NC * NP, D1), jnp.float32),
    mesh=_MESH,
    compiler_params=_SC_PARAMS,
    scratch_types=[
        pltpu.VMEM((G, B), jnp.int32),
        pltpu.VMEM((G, B), jnp.int32),
        pltpu.VMEM((B, D1), jnp.float32),
        pltpu.VMEM((B, D1), jnp.float32),
        pltpu.VMEM((ZB, D1), jnp.float32),
        pltpu.VMEM_SHARED((NP, D1), jnp.float32),
        pltpu.VMEM_SHARED((NP, D1), jnp.float32),
        pltpu.SemaphoreType.DMA,
        pltpu.SemaphoreType.DMA,
    ],
)
def _edge_kernel(edges_hbm, table_hbm, zeros_hbm, acc_hbm,
                 src_v, dst_v, rows0, rows1, zeros_v, acc_sh, table_sh,
                 sem0, sem1):
    c = lax.axis_index("c")
    s = lax.axis_index("s")
    wid = s * NC + c
    pltpu.sync_copy(edges_hbm.at[0, wid], src_v)
    pltpu.sync_copy(edges_hbm.at[1, wid], dst_v)
    pltpu.sync_copy(zeros_hbm, zeros_v)
    pltpu.sync_copy(table_hbm.at[pl.ds(s * RPT, RPT)],
                    table_sh.at[pl.ds(s * RPT, RPT)])

    def zero_body(j, carry):
        pltpu.sync_copy(zeros_v, acc_sh.at[pl.ds(s * RPT + j * ZB, ZB)])
        return carry

    lax.fori_loop(0, RPT // ZB, zero_body, 0)
    plsc.subcore_barrier()

    def wait_gather(buf, sem):
        pltpu.make_async_copy(table_hbm.at[pl.ds(0, B)], buf, sem).wait()

    pltpu.async_copy(table_sh.at[src_v.at[0]], rows0, sem0)

    def body(p, carry):
        g0 = 2 * p
        pltpu.async_copy(table_sh.at[src_v.at[g0 + 1]], rows1, sem1)
        wait_gather(rows0, sem0)
        pltpu.sync_copy(rows0, acc_sh.at[dst_v.at[g0]], add=True)

        @pl.when(p < G // 2 - 1)
        def _():
            pltpu.async_copy(table_sh.at[src_v.at[g0 + 2]], rows0, sem0)

        wait_gather(rows1, sem1)
        pltpu.sync_copy(rows1, acc_sh.at[dst_v.at[g0 + 1]], add=True)
        return carry

    lax.fori_loop(0, G // 2, body, 0)
    plsc.subcore_barrier()
    pltpu.sync_copy(acc_sh.at[pl.ds(s * RPT, RPT)],
                    acc_hbm.at[pl.ds(c * NP + s * RPT, RPT)])


_BLK = 1024
_BLK8 = _BLK // 8
NPP = NP // 8


def _dis_packed(deg_ref):
    dg = deg_ref[...]
    return lax.rsqrt(dg[0] + dg[1] + 1.0)


def _mask_packed(i):
    prow = i * _BLK8 + lax.broadcasted_iota(jnp.int32, (_BLK8, 128), 0)
    sub = lax.broadcasted_iota(jnp.int32, (_BLK8, 128), 1) // D1
    return prow * 8 + sub < N


_deg_spec = pl.BlockSpec((NC, _BLK8, 128), lambda i: (0, i, 0))
_pk_spec = pl.BlockSpec((_BLK8, 128), lambda i: (i, 0))
_acc_spec = pl.BlockSpec((NC, _BLK8, 128), lambda i: (0, i, 0))


def _tc1_body(deg_ref, xpk_ref, wbig_ref, out_ref):
    h = jnp.dot(xpk_ref[...], wbig_ref[...], preferred_element_type=jnp.float32)
    out_ref[...] = jnp.where(_mask_packed(pl.program_id(0)),
                             h * _dis_packed(deg_ref), 0.0)


_tc1 = pl.pallas_call(
    _tc1_body,
    grid=(NPP // _BLK8,),
    in_specs=[
        _deg_spec,
        pl.BlockSpec((_BLK8, 8 * D_IN), lambda i: (i, 0)),
        pl.BlockSpec((8 * D_IN, 128), lambda i: (0, 0)),
    ],
    out_specs=_pk_spec,
    out_shape=jax.ShapeDtypeStruct((NPP, 128), jnp.float32),
)


def _tc2_body(deg_ref, acc_ref, h1s_ref, b1_ref, out_ref):
    dis = _dis_packed(deg_ref)
    acc = acc_ref[0] + acc_ref[1]
    x2 = jnp.maximum(dis * (acc + h1s_ref[...]) + b1_ref[...], 0.0)
    out_ref[...] = jnp.where(_mask_packed(pl.program_id(0)), x2 * dis, 0.0)


_tc2 = pl.pallas_call(
    _tc2_body,
    grid=(NPP // _BLK8,),
    in_specs=[
        _deg_spec,
        _acc_spec,
        _pk_spec,
        pl.BlockSpec((1, 128), lambda i: (0, 0)),
    ],
    out_specs=_pk_spec,
    out_shape=jax.ShapeDtypeStruct((NPP, 128), jnp.float32),
)


def _tc3_body(deg_ref, acc_ref, t2_ref, w2big_ref, b2_ref, sum_ref, rep_ref, out_ref):
    dis = _dis_packed(deg_ref)
    aggp = dis * (acc_ref[0] + acc_ref[1] + t2_ref[...])
    z = jnp.dot(aggp, w2big_ref[...], preferred_element_type=jnp.float32) + b2_ref[...]
    m = jnp.max(z, axis=1, keepdims=True)
    e = jnp.exp(z - m)
    sums = jnp.dot(e, sum_ref[...], preferred_element_type=jnp.float32)
    lse = jnp.log(sums) + m
    out_ref[...] = z - jnp.dot(lse, rep_ref[...], preferred_element_type=jnp.float32)


_tc3 = pl.pallas_call(
    _tc3_body,
    grid=(NPP // _BLK8,),
    in_specs=[
        _deg_spec,
        _acc_spec,
        _pk_spec,
        pl.BlockSpec((128, 8 * DOUT), lambda i: (0, 0)),
        pl.BlockSpec((1, 8 * DOUT), lambda i: (0, 0)),
        pl.BlockSpec((8 * DOUT, 8), lambda i: (0, 0)),
        pl.BlockSpec((8, 8 * DOUT), lambda i: (0, 0)),
    ],
    out_specs=pl.BlockSpec((_BLK8, 8 * DOUT), lambda i: (i, 0)),
    out_shape=jax.ShapeDtypeStruct((N // 8, 8 * DOUT), jnp.float32),
)


def kernel(x, edge_index, W1, b1, W2, b2):
    edges = edge_index.reshape(2, NW, G, B)

    ones_b = jnp.ones((B, DW), jnp.float32)
    zeros_dw = jnp.zeros((ZB, DW), jnp.float32)
    zeros16 = jnp.zeros((ZB, D1), jnp.float32)

    xpk = x.reshape(N // 8, 8 * D_IN)
    w1big = jax.scipy.linalg.block_diag(*([W1] * 8))
    w2big = jax.scipy.linalg.block_diag(*([W2] * 8))
    b1p = jnp.tile(b1, 8).reshape(1, 128)
    b2p = jnp.tile(b2, 8).reshape(1, 8 * DOUT)
    sum_m = jnp.repeat(jnp.eye(8, dtype=jnp.float32), DOUT, axis=0)
    rep_m = sum_m.T

    degp = _deg_kernel(edges, ones_b, zeros_dw).reshape(NC, NPP, 128)
    h1s = _tc1(degp, xpk, w1big)
    acc1 = _edge_kernel(edges, h1s.reshape(NP, D1), zeros16).reshape(NC, NPP, 128)
    t2 = _tc2(degp, acc1, h1s, b1p)
    acc2 = _edge_kernel(edges, t2.reshape(NP, D1), zeros16).reshape(NC, NPP, 128)
    return _tc3(degp, acc2, t2, w2big, b2p, sum_m, rep_m).reshape(N, DOUT)
